# baseline (speedup 1.0000x reference)
import functools

import jax
import jax.numpy as jnp
from jax import lax
from jax.experimental import pallas as pl
from jax.experimental.pallas import tpu as pltpu
from jax.experimental.pallas import tpu_sc as plsc

B, D, T = 16, 256, 576
K = 8192
BK = 1024
NKB = K // BK


def _argmin_body(z_ref, emb_ref, idx_ref, minval_ref):
    kblk = pl.program_id(1)
    zb = z_ref[0]
    eb = emb_ref[...]
    mm = lax.dot_general(eb.astype(jnp.bfloat16), zb.astype(jnp.bfloat16),
                         (((1,), (0,)), ((), ())),
                         preferred_element_type=jnp.float32)
    en = jnp.sum(eb * eb, axis=1, keepdims=True)
    zn = jnp.sum(zb * zb, axis=0, keepdims=True)
    d = (zn + en) - 2.0 * mm
    lmin = jnp.min(d, axis=0, keepdims=True)
    ks = lax.broadcasted_iota(jnp.int32, d.shape, 0) + kblk * BK
    larg = jnp.min(jnp.where(d == lmin, ks, jnp.int32(2**31 - 1)),
                   axis=0, keepdims=True)

    @pl.when(kblk == 0)
    def _():
        minval_ref[...] = lmin
        idx_ref[0] = larg

    @pl.when(kblk > 0)
    def _():
        prev = minval_ref[...]
        better = lmin < prev
        minval_ref[...] = jnp.where(better, lmin, prev)
        idx_ref[0] = jnp.where(better, larg, idx_ref[0])


def _encode_indices(z, embedding):
    return pl.pallas_call(
        _argmin_body,
        grid=(B, NKB),
        in_specs=[
            pl.BlockSpec((1, D, T), lambda b, k: (b, 0, 0)),
            pl.BlockSpec((BK, D), lambda b, k: (k, 0)),
        ],
        out_specs=pl.BlockSpec((1, 1, T), lambda b, k: (b, 0, 0)),
        out_shape=jax.ShapeDtypeStruct((B, 1, T), jnp.int32),
        scratch_shapes=[pltpu.VMEM((1, T), jnp.float32)],
    )(z, embedding)


def _transpose_body(in_ref, out_ref):
    out_ref[0] = in_ref[0].T


def _transpose_btd(x):
    return pl.pallas_call(
        _transpose_body,
        grid=(B,),
        in_specs=[pl.BlockSpec((1, T, D), lambda b: (b, 0, 0))],
        out_specs=pl.BlockSpec((1, D, T), lambda b: (b, 0, 0)),
        out_shape=jax.ShapeDtypeStruct((B, D, T), jnp.float32),
    )(x)


def _make_sc_gather():
    info = plsc.get_sparse_core_info()
    nw = info.num_cores * info.num_subcores
    rows = B * T
    b_per_w = rows // nw
    mesh = plsc.VectorSubcoreMesh(core_axis_name="c", subcore_axis_name="s")

    @functools.partial(
        pl.kernel, mesh=mesh,
        out_type=jax.ShapeDtypeStruct((rows, D), jnp.float32),
        scratch_types=[
            pltpu.VMEM((b_per_w,), jnp.int32),
            pltpu.VMEM((b_per_w, D), jnp.float32),
            pltpu.SemaphoreType.DMA,
        ],
    )
    def gather_rows(table_hbm, idx_hbm, out_hbm, idx_v, rows_v, sem):
        wid = lax.axis_index("s") * info.num_cores + lax.axis_index("c")
        base = wid * b_per_w
        pltpu.sync_copy(idx_hbm.at[pl.ds(base, b_per_w)], idx_v)
        pltpu.async_copy(table_hbm.at[idx_v], rows_v, sem).wait()
        pltpu.sync_copy(rows_v, out_hbm.at[pl.ds(base, b_per_w)])

    return gather_rows


def kernel(z, embedding):
    idx = _encode_indices(z, embedding)
    idx_flat = idx.reshape(B * T)
    gathered = _make_sc_gather()(embedding, idx_flat)
    return _transpose_btd(gathered.reshape(B, T, D))


# speedup vs baseline: 1.0349x; 1.0349x over previous
import functools

import jax
import jax.numpy as jnp
from jax import lax
from jax.experimental import pallas as pl
from jax.experimental.pallas import tpu as pltpu
from jax.experimental.pallas import tpu_sc as plsc

B, D, T = 16, 256, 576
K = 8192
BK = 1024
NKB = K // BK


def _argmin_body(z_ref, emb_ref, idx_ref, minval_ref, zn_ref, zbb_ref,
                 en_ref, ebb_ref, kf_ref):
    b = pl.program_id(0)
    kblk = pl.program_id(1)

    @pl.when(jnp.logical_and(b == 0, kblk == 0))
    def _():
        kf_ref[...] = lax.broadcasted_iota(
            jnp.int32, (BK, T), 0).astype(jnp.float32)

    @pl.when(kblk == 0)
    def _():
        zb = z_ref[0]
        zn_ref[...] = jnp.sum(zb * zb, axis=0, keepdims=True)
        zbb_ref[...] = zb.astype(jnp.bfloat16)

    @pl.when(b == 0)
    def _():
        eb = emb_ref[...]
        en_ref[kblk] = jnp.sum(eb * eb, axis=1, keepdims=True)
        ebb_ref[kblk] = (eb * -2.0).astype(jnp.bfloat16)

    mm2 = lax.dot_general(ebb_ref[kblk], zbb_ref[...],
                          (((1,), (0,)), ((), ())),
                          preferred_element_type=jnp.float32)
    d = (zn_ref[...] + en_ref[kblk]) + mm2
    lmin = jnp.min(d, axis=0, keepdims=True)
    larg_f = jnp.min(jnp.where(d == lmin, kf_ref[...], jnp.float32(2.0 * BK)),
                     axis=0, keepdims=True)
    larg = larg_f.astype(jnp.int32) + kblk * BK

    @pl.when(kblk == 0)
    def _():
        minval_ref[...] = lmin
        idx_ref[0] = larg

    @pl.when(kblk > 0)
    def _():
        prev = minval_ref[...]
        better = lmin < prev
        minval_ref[...] = jnp.where(better, lmin, prev)
        idx_ref[0] = jnp.where(better, larg, idx_ref[0])


def _encode_indices(z, embedding):
    return pl.pallas_call(
        _argmin_body,
        grid=(B, NKB),
        in_specs=[
            pl.BlockSpec((1, D, T), lambda b, k: (b, 0, 0)),
            pl.BlockSpec((BK, D), lambda b, k: (k, 0)),
        ],
        out_specs=pl.BlockSpec((1, 1, T), lambda b, k: (b, 0, 0)),
        out_shape=jax.ShapeDtypeStruct((B, 1, T), jnp.int32),
        scratch_shapes=[
            pltpu.VMEM((1, T), jnp.float32),
            pltpu.VMEM((1, T), jnp.float32),
            pltpu.VMEM((D, T), jnp.bfloat16),
            pltpu.VMEM((NKB, BK, 1), jnp.float32),
            pltpu.VMEM((NKB, BK, D), jnp.bfloat16),
            pltpu.VMEM((BK, T), jnp.float32),
        ],
    )(z, embedding)


def _transpose_body(in_ref, out_ref):
    out_ref[0] = in_ref[0].T


def _transpose_btd(x):
    return pl.pallas_call(
        _transpose_body,
        grid=(B,),
        in_specs=[pl.BlockSpec((1, T, D), lambda b: (b, 0, 0))],
        out_specs=pl.BlockSpec((1, D, T), lambda b: (b, 0, 0)),
        out_shape=jax.ShapeDtypeStruct((B, D, T), jnp.float32),
    )(x)


def _make_sc_gather():
    info = plsc.get_sparse_core_info()
    nw = info.num_cores * info.num_subcores
    rows = B * T
    b_per_w = rows // nw
    mesh = plsc.VectorSubcoreMesh(core_axis_name="c", subcore_axis_name="s")

    @functools.partial(
        pl.kernel, mesh=mesh,
        out_type=jax.ShapeDtypeStruct((rows, D), jnp.float32),
        scratch_types=[
            pltpu.VMEM((b_per_w,), jnp.int32),
            pltpu.VMEM((b_per_w, D), jnp.float32),
            pltpu.SemaphoreType.DMA,
        ],
    )
    def gather_rows(table_hbm, idx_hbm, out_hbm, idx_v, rows_v, sem):
        wid = lax.axis_index("s") * info.num_cores + lax.axis_index("c")
        base = wid * b_per_w
        pltpu.sync_copy(idx_hbm.at[pl.ds(base, b_per_w)], idx_v)
        pltpu.async_copy(table_hbm.at[idx_v], rows_v, sem).wait()
        pltpu.sync_copy(rows_v, out_hbm.at[pl.ds(base, b_per_w)])

    return gather_rows


def kernel(z, embedding):
    idx = _encode_indices(z, embedding)
    idx_flat = idx.reshape(B * T)
    gathered = _make_sc_gather()(embedding, idx_flat)
    return _transpose_btd(gathered.reshape(B, T, D))
